# baseline (speedup 1.0000x reference)
import functools

import jax
import jax.numpy as jnp
from jax import lax
from jax.experimental import pallas as pl
from jax.experimental.pallas import tpu as pltpu
from jax.experimental.pallas import tpu_sc as plsc

B, N, D = 32, 256, 384
CODE_DIM, N_CODES, N_SLOTS = 64, 8192, 8


def _idx_kernel(vmask_ref, V_ref, xn_ref, ln1_g_ref, ln1_b_ref, W1_ref, b1_ref,
                W2_ref, b2_ref, mu_ref, idx_ref, mun_ref):
    @pl.when(pl.program_id(0) == 0)
    def _():
        mu = mu_ref[...]
        norm = jnp.sqrt(jnp.sum(mu * mu, axis=1, keepdims=True))
        mun_ref[...] = mu / jnp.maximum(norm, 1e-8)

    Vc = V_ref[0]
    xn = xn_ref[0]
    frac_pos = jnp.mean((Vc > 0).astype(jnp.float32), axis=0, keepdims=True)
    sign = jnp.where(frac_pos < 0.5, -1.0, 1.0)
    scale = sign * vmask_ref[...]
    Vs = Vc * scale
    proj = lax.dot_general(Vs, xn, (((0,), (0,)), ((), ())),
                           preferred_element_type=jnp.float32)
    m = jnp.mean(proj, axis=-1, keepdims=True)
    v = jnp.mean((proj - m) ** 2, axis=-1, keepdims=True)
    h = (proj - m) / jnp.sqrt(v + 1e-5) * ln1_g_ref[...] + ln1_b_ref[...]
    h = jnp.maximum(
        lax.dot_general(h, W1_ref[...], (((1,), (0,)), ((), ())),
                        preferred_element_type=jnp.float32) + b1_ref[...], 0.0)
    z = lax.dot_general(h, W2_ref[...], (((1,), (0,)), ((), ())),
                        preferred_element_type=jnp.float32) + b2_ref[...]
    zn = z / jnp.maximum(jnp.sqrt(jnp.sum(z * z, axis=-1, keepdims=True)), 1e-8)
    sim = lax.dot_general(zn, mun_ref[...], (((1,), (1,)), ((), ())),
                          preferred_element_type=jnp.float32)
    idx_ref[0, 0, :] = jnp.argmax(sim, axis=-1).astype(jnp.int32)


def _up_kernel(muP_ref, lsP_ref, par_ref, noise_ref, W3_ref, b3_ref, W4_ref,
               b4_ref, ln2_g_ref, ln2_b_ref, out_ref):
    par = par_ref[...] > 0.5
    muP = muP_ref[...]
    lsP = lsP_ref[...]
    mu_s = jnp.where(par, muP[:, CODE_DIM:], muP[:, :CODE_DIM])
    ls_s = jnp.where(par, lsP[:, CODE_DIM:], lsP[:, :CODE_DIM])
    sample = mu_s + jnp.exp(ls_s) * noise_ref[...]
    u = jnp.maximum(
        lax.dot_general(sample, W3_ref[...], (((1,), (0,)), ((), ())),
                        preferred_element_type=jnp.float32) + b3_ref[...], 0.0)
    u = lax.dot_general(u, W4_ref[...], (((1,), (0,)), ((), ())),
                        preferred_element_type=jnp.float32) + b4_ref[...]
    m = jnp.mean(u, axis=-1, keepdims=True)
    v = jnp.mean((u - m) ** 2, axis=-1, keepdims=True)
    out_ref[...] = (u - m) / jnp.sqrt(v + 1e-5) * ln2_g_ref[...] + ln2_b_ref[...]


def _sc_gather(muP, lsP, idx_pair):
    width = muP.shape[1]
    nrows = idx_pair.shape[0]
    info = plsc.get_sparse_core_info()
    nc, ns = info.num_cores, info.num_subcores
    rows_per_w = nrows // (nc * ns)
    mesh = plsc.VectorSubcoreMesh(core_axis_name="c", subcore_axis_name="s")

    @functools.partial(
        pl.kernel, mesh=mesh,
        out_type=(
            jax.ShapeDtypeStruct((nrows, width), jnp.float32),
            jax.ShapeDtypeStruct((nrows, width), jnp.float32),
        ),
        scratch_types=[
            pltpu.VMEM((rows_per_w,), jnp.int32),
            pltpu.VMEM((rows_per_w, width), jnp.float32),
            pltpu.VMEM((rows_per_w, width), jnp.float32),
            pltpu.SemaphoreType.DMA,
            pltpu.SemaphoreType.DMA,
        ],
    )
    def k(mu_hbm, ls_hbm, idx_hbm, mu_out, ls_out, idx_v, mu_v, ls_v, s1, s2):
        wid = lax.axis_index("s") * nc + lax.axis_index("c")
        base = wid * rows_per_w
        pltpu.sync_copy(idx_hbm.at[pl.ds(base, rows_per_w)], idx_v)
        cp1 = pltpu.async_copy(mu_hbm.at[idx_v], mu_v, s1)
        cp2 = pltpu.async_copy(ls_hbm.at[idx_v], ls_v, s2)
        cp1.wait()
        cp2.wait()
        pltpu.sync_copy(mu_v, mu_out.at[pl.ds(base, rows_per_w)])
        pltpu.sync_copy(ls_v, ls_out.at[pl.ds(base, rows_per_w)])

    return k(muP, lsP, idx_pair)


def kernel(x, n_slots, mu, log_sigma, ln1_g, ln1_b, W1, b1, W2, b2, W3, b3,
           W4, b4, ln2_g, ln2_b, noise):
    xn = x / jnp.maximum(jnp.linalg.norm(x, axis=-1, keepdims=True), 1e-12)
    cov = jnp.einsum('bnd,bmd->bnm', xn, xn)
    _, eig_vectors = jnp.linalg.eigh(cov)

    slots = noise.shape[1]
    vmask = jnp.flip((jnp.arange(slots) < n_slots).astype(jnp.float32),
                     axis=0).reshape(1, slots)
    Vc8 = lax.slice(eig_vectors, (0, 0, N - slots), (B, N, N))

    idx = pl.pallas_call(
        _idx_kernel,
        grid=(B,),
        in_specs=[
            pl.BlockSpec((1, slots), lambda b: (0, 0)),
            pl.BlockSpec((1, N, slots), lambda b: (b, 0, 0)),
            pl.BlockSpec((1, N, D), lambda b: (b, 0, 0)),
            pl.BlockSpec((1, D), lambda b: (0, 0)),
            pl.BlockSpec((1, D), lambda b: (0, 0)),
            pl.BlockSpec((D, D), lambda b: (0, 0)),
            pl.BlockSpec((1, D), lambda b: (0, 0)),
            pl.BlockSpec((D, CODE_DIM), lambda b: (0, 0)),
            pl.BlockSpec((1, CODE_DIM), lambda b: (0, 0)),
            pl.BlockSpec((N_CODES, CODE_DIM), lambda b: (0, 0)),
        ],
        out_specs=pl.BlockSpec((1, 1, slots), lambda b: (b, 0, 0)),
        out_shape=jax.ShapeDtypeStruct((B, 1, slots), jnp.int32),
        scratch_shapes=[pltpu.VMEM((N_CODES, CODE_DIM), jnp.float32)],
    )(vmask, Vc8, xn, ln1_g.reshape(1, D), ln1_b.reshape(1, D),
      W1, b1.reshape(1, D), W2, b2.reshape(1, CODE_DIM), mu)

    idx_flat = jnp.flip(idx.reshape(B, slots), axis=1).reshape(B * slots)
    parity = (idx_flat & 1).astype(jnp.float32).reshape(B * slots, 1)

    muP, lsP = _sc_gather(mu.reshape(N_CODES // 2, 2 * CODE_DIM),
                          log_sigma.reshape(N_CODES // 2, 2 * CODE_DIM),
                          idx_flat >> 1)

    out = pl.pallas_call(
        _up_kernel,
        in_specs=[
            pl.BlockSpec((B * slots, 2 * CODE_DIM), lambda: (0, 0)),
            pl.BlockSpec((B * slots, 2 * CODE_DIM), lambda: (0, 0)),
            pl.BlockSpec((B * slots, 1), lambda: (0, 0)),
            pl.BlockSpec((B * slots, CODE_DIM), lambda: (0, 0)),
            pl.BlockSpec((CODE_DIM, D), lambda: (0, 0)),
            pl.BlockSpec((1, D), lambda: (0, 0)),
            pl.BlockSpec((D, D), lambda: (0, 0)),
            pl.BlockSpec((1, D), lambda: (0, 0)),
            pl.BlockSpec((1, D), lambda: (0, 0)),
            pl.BlockSpec((1, D), lambda: (0, 0)),
        ],
        out_specs=pl.BlockSpec((B * slots, D), lambda: (0, 0)),
        out_shape=jax.ShapeDtypeStruct((B * slots, D), jnp.float32),
    )(muP, lsP, parity, noise.reshape(B * slots, CODE_DIM), W3,
      b3.reshape(1, D), W4, b4.reshape(1, D), ln2_g.reshape(1, D),
      ln2_b.reshape(1, D))

    return out.reshape(B, slots, D)


# speedup vs baseline: 1.0000x; 1.0000x over previous
import functools

import jax
import jax.numpy as jnp
from jax import lax
from jax.experimental import pallas as pl
from jax.experimental.pallas import tpu as pltpu
from jax.experimental.pallas import tpu_sc as plsc

B, N, D = 32, 256, 384
CODE_DIM, N_CODES, N_SLOTS = 64, 8192, 8


def _idx_kernel(vmask_ref, V_ref, xn_ref, ln1_g_ref, ln1_b_ref, W1_ref, b1_ref,
                W2_ref, b2_ref, mu_ref, idx_ref, mun_ref):
    @pl.when(pl.program_id(0) == 0)
    def _():
        mu = mu_ref[...]
        norm = jnp.sqrt(jnp.sum(mu * mu, axis=1, keepdims=True))
        mun_ref[...] = mu / jnp.maximum(norm, 1e-8)

    Vc = V_ref[0][:, N - 8:]
    xn = xn_ref[0]
    frac_pos = jnp.mean((Vc > 0).astype(jnp.float32), axis=0, keepdims=True)
    sign = jnp.where(frac_pos < 0.5, -1.0, 1.0)
    scale = sign * vmask_ref[...]
    Vs = Vc * scale
    proj = lax.dot_general(Vs, xn, (((0,), (0,)), ((), ())),
                           preferred_element_type=jnp.float32)
    m = jnp.mean(proj, axis=-1, keepdims=True)
    v = jnp.mean((proj - m) ** 2, axis=-1, keepdims=True)
    h = (proj - m) / jnp.sqrt(v + 1e-5) * ln1_g_ref[...] + ln1_b_ref[...]
    h = jnp.maximum(
        lax.dot_general(h, W1_ref[...], (((1,), (0,)), ((), ())),
                        preferred_element_type=jnp.float32) + b1_ref[...], 0.0)
    z = lax.dot_general(h, W2_ref[...], (((1,), (0,)), ((), ())),
                        preferred_element_type=jnp.float32) + b2_ref[...]
    zn = z / jnp.maximum(jnp.sqrt(jnp.sum(z * z, axis=-1, keepdims=True)), 1e-8)
    sim = lax.dot_general(zn, mun_ref[...], (((1,), (1,)), ((), ())),
                          preferred_element_type=jnp.float32)
    idx_ref[0, 0, :] = jnp.argmax(sim, axis=-1).astype(jnp.int32)


def _up_kernel(muP_ref, lsP_ref, par_ref, noise_ref, W3_ref, b3_ref, W4_ref,
               b4_ref, ln2_g_ref, ln2_b_ref, out_ref):
    par = par_ref[...] > 0.5
    muP = muP_ref[...]
    lsP = lsP_ref[...]
    mu_s = jnp.where(par, muP[:, CODE_DIM:], muP[:, :CODE_DIM])
    ls_s = jnp.where(par, lsP[:, CODE_DIM:], lsP[:, :CODE_DIM])
    sample = mu_s + jnp.exp(ls_s) * noise_ref[...]
    u = jnp.maximum(
        lax.dot_general(sample, W3_ref[...], (((1,), (0,)), ((), ())),
                        preferred_element_type=jnp.float32) + b3_ref[...], 0.0)
    u = lax.dot_general(u, W4_ref[...], (((1,), (0,)), ((), ())),
                        preferred_element_type=jnp.float32) + b4_ref[...]
    m = jnp.mean(u, axis=-1, keepdims=True)
    v = jnp.mean((u - m) ** 2, axis=-1, keepdims=True)
    out_ref[...] = (u - m) / jnp.sqrt(v + 1e-5) * ln2_g_ref[...] + ln2_b_ref[...]


def _sc_gather(muP, lsP, idx_pair):
    width = muP.shape[1]
    nrows = idx_pair.shape[0]
    info = plsc.get_sparse_core_info()
    nc, ns = info.num_cores, info.num_subcores
    rows_per_w = nrows // (nc * ns)
    mesh = plsc.VectorSubcoreMesh(core_axis_name="c", subcore_axis_name="s")

    @functools.partial(
        pl.kernel, mesh=mesh,
        out_type=(
            jax.ShapeDtypeStruct((nrows, width), jnp.float32),
            jax.ShapeDtypeStruct((nrows, width), jnp.float32),
        ),
        scratch_types=[
            pltpu.VMEM((rows_per_w,), jnp.int32),
            pltpu.VMEM((rows_per_w, width), jnp.float32),
            pltpu.VMEM((rows_per_w, width), jnp.float32),
            pltpu.SemaphoreType.DMA,
            pltpu.SemaphoreType.DMA,
        ],
    )
    def k(mu_hbm, ls_hbm, idx_hbm, mu_out, ls_out, idx_v, mu_v, ls_v, s1, s2):
        wid = lax.axis_index("s") * nc + lax.axis_index("c")
        base = wid * rows_per_w
        pltpu.sync_copy(idx_hbm.at[pl.ds(base, rows_per_w)], idx_v)
        cp1 = pltpu.async_copy(mu_hbm.at[idx_v], mu_v, s1)
        cp2 = pltpu.async_copy(ls_hbm.at[idx_v], ls_v, s2)
        cp1.wait()
        cp2.wait()
        pltpu.sync_copy(mu_v, mu_out.at[pl.ds(base, rows_per_w)])
        pltpu.sync_copy(ls_v, ls_out.at[pl.ds(base, rows_per_w)])

    return k(muP, lsP, idx_pair)


def kernel(x, n_slots, mu, log_sigma, ln1_g, ln1_b, W1, b1, W2, b2, W3, b3,
           W4, b4, ln2_g, ln2_b, noise):
    xn = x / jnp.maximum(jnp.linalg.norm(x, axis=-1, keepdims=True), 1e-12)
    cov = jnp.einsum('bnd,bmd->bnm', xn, xn)
    _, eig_vectors = jnp.linalg.eigh(cov)

    slots = noise.shape[1]
    vmask = jnp.flip((jnp.arange(slots) < n_slots).astype(jnp.float32),
                     axis=0).reshape(1, slots)
    idx = pl.pallas_call(
        _idx_kernel,
        grid=(B,),
        in_specs=[
            pl.BlockSpec((1, slots), lambda b: (0, 0)),
            pl.BlockSpec((1, N, N), lambda b: (b, 0, 0)),
            pl.BlockSpec((1, N, D), lambda b: (b, 0, 0)),
            pl.BlockSpec((1, D), lambda b: (0, 0)),
            pl.BlockSpec((1, D), lambda b: (0, 0)),
            pl.BlockSpec((D, D), lambda b: (0, 0)),
            pl.BlockSpec((1, D), lambda b: (0, 0)),
            pl.BlockSpec((D, CODE_DIM), lambda b: (0, 0)),
            pl.BlockSpec((1, CODE_DIM), lambda b: (0, 0)),
            pl.BlockSpec((N_CODES, CODE_DIM), lambda b: (0, 0)),
        ],
        out_specs=pl.BlockSpec((1, 1, slots), lambda b: (b, 0, 0)),
        out_shape=jax.ShapeDtypeStruct((B, 1, slots), jnp.int32),
        scratch_shapes=[pltpu.VMEM((N_CODES, CODE_DIM), jnp.float32)],
    )(vmask, eig_vectors, xn, ln1_g.reshape(1, D), ln1_b.reshape(1, D),
      W1, b1.reshape(1, D), W2, b2.reshape(1, CODE_DIM), mu)

    idx_flat = jnp.flip(idx.reshape(B, slots), axis=1).reshape(B * slots)
    parity = (idx_flat & 1).astype(jnp.float32).reshape(B * slots, 1)

    muP, lsP = _sc_gather(mu.reshape(N_CODES // 2, 2 * CODE_DIM),
                          log_sigma.reshape(N_CODES // 2, 2 * CODE_DIM),
                          idx_flat >> 1)

    out = pl.pallas_call(
        _up_kernel,
        in_specs=[
            pl.BlockSpec((B * slots, 2 * CODE_DIM), lambda: (0, 0)),
            pl.BlockSpec((B * slots, 2 * CODE_DIM), lambda: (0, 0)),
            pl.BlockSpec((B * slots, 1), lambda: (0, 0)),
            pl.BlockSpec((B * slots, CODE_DIM), lambda: (0, 0)),
            pl.BlockSpec((CODE_DIM, D), lambda: (0, 0)),
            pl.BlockSpec((1, D), lambda: (0, 0)),
            pl.BlockSpec((D, D), lambda: (0, 0)),
            pl.BlockSpec((1, D), lambda: (0, 0)),
            pl.BlockSpec((1, D), lambda: (0, 0)),
            pl.BlockSpec((1, D), lambda: (0, 0)),
        ],
        out_specs=pl.BlockSpec((B * slots, D), lambda: (0, 0)),
        out_shape=jax.ShapeDtypeStruct((B * slots, D), jnp.float32),
    )(muP, lsP, parity, noise.reshape(B * slots, CODE_DIM), W3,
      b3.reshape(1, D), W4, b4.reshape(1, D), ln2_g.reshape(1, D),
      ln2_b.reshape(1, D))

    return out.reshape(B, slots, D)


# speedup vs baseline: 1.0010x; 1.0009x over previous
import functools

import jax
import jax.numpy as jnp
from jax import lax
from jax.experimental import pallas as pl
from jax.experimental.pallas import tpu as pltpu
from jax.experimental.pallas import tpu_sc as plsc

B, N, D = 32, 256, 384
CODE_DIM, N_CODES, N_SLOTS = 64, 8192, 8


def _idx_kernel(vmask_ref, V_ref, xn_ref, ln1_g_ref, ln1_b_ref, W1_ref, b1_ref,
                W2_ref, b2_ref, mu_ref, idx_ref, proj_ref):
    def body(b, _):
        Vc = V_ref[b, :, N - 8:]
        frac_pos = jnp.mean((Vc > 0).astype(jnp.float32), axis=0, keepdims=True)
        sign = jnp.where(frac_pos < 0.5, -1.0, 1.0)
        scale = sign * vmask_ref[...]
        proj_ref[pl.ds(b * 8, 8), :] = lax.dot_general(
            Vc * scale, xn_ref[b], (((0,), (0,)), ((), ())),
            preferred_element_type=jnp.float32)
        return 0

    lax.fori_loop(0, B, body, 0, unroll=False)

    proj = proj_ref[...]
    m = jnp.mean(proj, axis=-1, keepdims=True)
    v = jnp.mean((proj - m) ** 2, axis=-1, keepdims=True)
    h = (proj - m) / jnp.sqrt(v + 1e-5) * ln1_g_ref[...] + ln1_b_ref[...]
    h = jnp.maximum(
        lax.dot_general(h, W1_ref[...], (((1,), (0,)), ((), ())),
                        preferred_element_type=jnp.float32) + b1_ref[...], 0.0)
    z = lax.dot_general(h, W2_ref[...], (((1,), (0,)), ((), ())),
                        preferred_element_type=jnp.float32) + b2_ref[...]
    zn = z / jnp.maximum(jnp.sqrt(jnp.sum(z * z, axis=-1, keepdims=True)), 1e-8)
    mu = mu_ref[...]
    norm = jnp.sqrt(jnp.sum(mu * mu, axis=1, keepdims=True))
    mun = mu / jnp.maximum(norm, 1e-8)
    sim = lax.dot_general(zn, mun, (((1,), (1,)), ((), ())),
                          preferred_element_type=jnp.float32)
    idx_ref[...] = jnp.argmax(sim, axis=-1).astype(jnp.int32).reshape(1, B * 8)


def _up_kernel(muP_ref, lsP_ref, par_ref, noise_ref, W3_ref, b3_ref, W4_ref,
               b4_ref, ln2_g_ref, ln2_b_ref, out_ref):
    par = par_ref[...] > 0.5
    muP = muP_ref[...]
    lsP = lsP_ref[...]
    mu_s = jnp.where(par, muP[:, CODE_DIM:], muP[:, :CODE_DIM])
    ls_s = jnp.where(par, lsP[:, CODE_DIM:], lsP[:, :CODE_DIM])
    sample = mu_s + jnp.exp(ls_s) * noise_ref[...]
    u = jnp.maximum(
        lax.dot_general(sample, W3_ref[...], (((1,), (0,)), ((), ())),
                        preferred_element_type=jnp.float32) + b3_ref[...], 0.0)
    u = lax.dot_general(u, W4_ref[...], (((1,), (0,)), ((), ())),
                        preferred_element_type=jnp.float32) + b4_ref[...]
    m = jnp.mean(u, axis=-1, keepdims=True)
    v = jnp.mean((u - m) ** 2, axis=-1, keepdims=True)
    out_ref[...] = (u - m) / jnp.sqrt(v + 1e-5) * ln2_g_ref[...] + ln2_b_ref[...]


def _sc_gather(muP, lsP, idx_pair):
    width = muP.shape[1]
    nrows = idx_pair.shape[0]
    info = plsc.get_sparse_core_info()
    nc, ns = info.num_cores, info.num_subcores
    rows_per_w = nrows // (nc * ns)
    mesh = plsc.VectorSubcoreMesh(core_axis_name="c", subcore_axis_name="s")

    @functools.partial(
        pl.kernel, mesh=mesh,
        out_type=(
            jax.ShapeDtypeStruct((nrows, width), jnp.float32),
            jax.ShapeDtypeStruct((nrows, width), jnp.float32),
        ),
        scratch_types=[
            pltpu.VMEM((rows_per_w,), jnp.int32),
            pltpu.VMEM((rows_per_w, width), jnp.float32),
            pltpu.VMEM((rows_per_w, width), jnp.float32),
            pltpu.SemaphoreType.DMA,
            pltpu.SemaphoreType.DMA,
        ],
    )
    def k(mu_hbm, ls_hbm, idx_hbm, mu_out, ls_out, idx_v, mu_v, ls_v, s1, s2):
        wid = lax.axis_index("s") * nc + lax.axis_index("c")
        base = wid * rows_per_w
        pltpu.sync_copy(idx_hbm.at[pl.ds(base, rows_per_w)], idx_v)
        cp1 = pltpu.async_copy(mu_hbm.at[idx_v], mu_v, s1)
        cp2 = pltpu.async_copy(ls_hbm.at[idx_v], ls_v, s2)
        cp1.wait()
        cp2.wait()
        pltpu.sync_copy(mu_v, mu_out.at[pl.ds(base, rows_per_w)])
        pltpu.sync_copy(ls_v, ls_out.at[pl.ds(base, rows_per_w)])

    return k(muP, lsP, idx_pair)


def kernel(x, n_slots, mu, log_sigma, ln1_g, ln1_b, W1, b1, W2, b2, W3, b3,
           W4, b4, ln2_g, ln2_b, noise):
    xn = x / jnp.maximum(jnp.linalg.norm(x, axis=-1, keepdims=True), 1e-12)
    cov = jnp.einsum('bnd,bmd->bnm', xn, xn)
    _, eig_vectors = jnp.linalg.eigh(cov)

    slots = noise.shape[1]
    vmask = jnp.flip((jnp.arange(slots) < n_slots).astype(jnp.float32),
                     axis=0).reshape(1, slots)
    idx = pl.pallas_call(
        _idx_kernel,
        in_specs=[
            pl.BlockSpec((1, slots), lambda: (0, 0)),
            pl.BlockSpec((B, N, N), lambda: (0, 0, 0)),
            pl.BlockSpec((B, N, D), lambda: (0, 0, 0)),
            pl.BlockSpec((1, D), lambda: (0, 0)),
            pl.BlockSpec((1, D), lambda: (0, 0)),
            pl.BlockSpec((D, D), lambda: (0, 0)),
            pl.BlockSpec((1, D), lambda: (0, 0)),
            pl.BlockSpec((D, CODE_DIM), lambda: (0, 0)),
            pl.BlockSpec((1, CODE_DIM), lambda: (0, 0)),
            pl.BlockSpec((N_CODES, CODE_DIM), lambda: (0, 0)),
        ],
        out_specs=pl.BlockSpec((1, B * slots), lambda: (0, 0)),
        out_shape=jax.ShapeDtypeStruct((1, B * slots), jnp.int32),
        scratch_shapes=[pltpu.VMEM((B * slots, D), jnp.float32)],
    )(vmask, eig_vectors, xn, ln1_g.reshape(1, D), ln1_b.reshape(1, D),
      W1, b1.reshape(1, D), W2, b2.reshape(1, CODE_DIM), mu)

    idx_flat = jnp.flip(idx.reshape(B, slots), axis=1).reshape(B * slots)
    parity = (idx_flat & 1).astype(jnp.float32).reshape(B * slots, 1)

    muP, lsP = _sc_gather(mu.reshape(N_CODES // 2, 2 * CODE_DIM),
                          log_sigma.reshape(N_CODES // 2, 2 * CODE_DIM),
                          idx_flat >> 1)

    out = pl.pallas_call(
        _up_kernel,
        in_specs=[
            pl.BlockSpec((B * slots, 2 * CODE_DIM), lambda: (0, 0)),
            pl.BlockSpec((B * slots, 2 * CODE_DIM), lambda: (0, 0)),
            pl.BlockSpec((B * slots, 1), lambda: (0, 0)),
            pl.BlockSpec((B * slots, CODE_DIM), lambda: (0, 0)),
            pl.BlockSpec((CODE_DIM, D), lambda: (0, 0)),
            pl.BlockSpec((1, D), lambda: (0, 0)),
            pl.BlockSpec((D, D), lambda: (0, 0)),
            pl.BlockSpec((1, D), lambda: (0, 0)),
            pl.BlockSpec((1, D), lambda: (0, 0)),
            pl.BlockSpec((1, D), lambda: (0, 0)),
        ],
        out_specs=pl.BlockSpec((B * slots, D), lambda: (0, 0)),
        out_shape=jax.ShapeDtypeStruct((B * slots, D), jnp.float32),
    )(muP, lsP, parity, noise.reshape(B * slots, CODE_DIM), W3,
      b3.reshape(1, D), W4, b4.reshape(1, D), ln2_g.reshape(1, D),
      ln2_b.reshape(1, D))

    return out.reshape(B, slots, D)


# speedup vs baseline: 1.0012x; 1.0003x over previous
import functools

import jax
import jax.numpy as jnp
from jax import lax
from jax.experimental import pallas as pl
from jax.experimental.pallas import tpu as pltpu
from jax.experimental.pallas import tpu_sc as plsc

B, N, D = 32, 256, 384
CODE_DIM, N_CODES, N_SLOTS = 64, 8192, 8


def _idx_kernel(vmask_ref, V_ref, xn_ref, ln1_g_ref, ln1_b_ref, W1_ref, b1_ref,
                W2_ref, b2_ref, mu_ref, idx_ref, proj_ref):
    def body(b, _):
        Vc = V_ref[b, :, 120:]
        frac_pos = jnp.mean((Vc > 0).astype(jnp.float32), axis=0, keepdims=True)
        sign = jnp.where(frac_pos < 0.5, -1.0, 1.0)
        scale = sign * vmask_ref[...]
        proj_ref[pl.ds(b * 8, 8), :] = lax.dot_general(
            Vc * scale, xn_ref[b], (((0,), (0,)), ((), ())),
            preferred_element_type=jnp.float32)
        return 0

    lax.fori_loop(0, B, body, 0, unroll=True)

    proj = proj_ref[...]
    m = jnp.mean(proj, axis=-1, keepdims=True)
    v = jnp.mean((proj - m) ** 2, axis=-1, keepdims=True)
    h = (proj - m) / jnp.sqrt(v + 1e-5) * ln1_g_ref[...] + ln1_b_ref[...]
    h = jnp.maximum(
        lax.dot_general(h, W1_ref[...], (((1,), (0,)), ((), ())),
                        preferred_element_type=jnp.float32) + b1_ref[...], 0.0)
    z = lax.dot_general(h, W2_ref[...], (((1,), (0,)), ((), ())),
                        preferred_element_type=jnp.float32) + b2_ref[...]
    zn = z / jnp.maximum(jnp.sqrt(jnp.sum(z * z, axis=-1, keepdims=True)), 1e-8)
    mu = mu_ref[...]
    norm = jnp.sqrt(jnp.sum(mu * mu, axis=1, keepdims=True))
    mun = mu / jnp.maximum(norm, 1e-8)
    sim = lax.dot_general(zn, mun, (((1,), (1,)), ((), ())),
                          preferred_element_type=jnp.float32)
    idx_ref[...] = jnp.argmax(sim, axis=-1).astype(jnp.int32).reshape(1, B * 8)


def _up_kernel(muP_ref, lsP_ref, par_ref, noise_ref, W3_ref, b3_ref, W4_ref,
               b4_ref, ln2_g_ref, ln2_b_ref, out_ref):
    par = par_ref[...] > 0.5
    muP = muP_ref[...]
    lsP = lsP_ref[...]
    mu_s = jnp.where(par, muP[:, CODE_DIM:], muP[:, :CODE_DIM])
    ls_s = jnp.where(par, lsP[:, CODE_DIM:], lsP[:, :CODE_DIM])
    sample = mu_s + jnp.exp(ls_s) * noise_ref[...]
    u = jnp.maximum(
        lax.dot_general(sample, W3_ref[...], (((1,), (0,)), ((), ())),
                        preferred_element_type=jnp.float32) + b3_ref[...], 0.0)
    u = lax.dot_general(u, W4_ref[...], (((1,), (0,)), ((), ())),
                        preferred_element_type=jnp.float32) + b4_ref[...]
    m = jnp.mean(u, axis=-1, keepdims=True)
    v = jnp.mean((u - m) ** 2, axis=-1, keepdims=True)
    out_ref[...] = (u - m) / jnp.sqrt(v + 1e-5) * ln2_g_ref[...] + ln2_b_ref[...]


def _sc_gather(muP, lsP, idx_pair):
    width = muP.shape[1]
    nrows = idx_pair.shape[0]
    info = plsc.get_sparse_core_info()
    nc, ns = info.num_cores, info.num_subcores
    rows_per_w = nrows // (nc * ns)
    mesh = plsc.VectorSubcoreMesh(core_axis_name="c", subcore_axis_name="s")

    @functools.partial(
        pl.kernel, mesh=mesh,
        out_type=(
            jax.ShapeDtypeStruct((nrows, width), jnp.float32),
            jax.ShapeDtypeStruct((nrows, width), jnp.float32),
        ),
        scratch_types=[
            pltpu.VMEM((rows_per_w,), jnp.int32),
            pltpu.VMEM((rows_per_w, width), jnp.float32),
            pltpu.VMEM((rows_per_w, width), jnp.float32),
            pltpu.SemaphoreType.DMA,
            pltpu.SemaphoreType.DMA,
        ],
    )
    def k(mu_hbm, ls_hbm, idx_hbm, mu_out, ls_out, idx_v, mu_v, ls_v, s1, s2):
        wid = lax.axis_index("s") * nc + lax.axis_index("c")
        base = wid * rows_per_w
        pltpu.sync_copy(idx_hbm.at[pl.ds(base, rows_per_w)], idx_v)
        cp1 = pltpu.async_copy(mu_hbm.at[idx_v], mu_v, s1)
        cp2 = pltpu.async_copy(ls_hbm.at[idx_v], ls_v, s2)
        cp1.wait()
        cp2.wait()
        pltpu.sync_copy(mu_v, mu_out.at[pl.ds(base, rows_per_w)])
        pltpu.sync_copy(ls_v, ls_out.at[pl.ds(base, rows_per_w)])

    return k(muP, lsP, idx_pair)


def kernel(x, n_slots, mu, log_sigma, ln1_g, ln1_b, W1, b1, W2, b2, W3, b3,
           W4, b4, ln2_g, ln2_b, noise):
    xn = x / jnp.maximum(jnp.linalg.norm(x, axis=-1, keepdims=True), 1e-12)
    cov = jnp.einsum('bnd,bmd->bnm', xn, xn)
    _, eig_vectors = jnp.linalg.eigh(cov)

    slots = noise.shape[1]
    vmask = jnp.flip((jnp.arange(slots) < n_slots).astype(jnp.float32),
                     axis=0).reshape(1, slots)
    idx = pl.pallas_call(
        _idx_kernel,
        grid=(1,),
        in_specs=[
            pl.BlockSpec((1, slots), lambda g: (0, 0)),
            pl.BlockSpec((B, N, 128), lambda g: (0, 0, N // 128 - 1)),
            pl.BlockSpec((B, N, D), lambda g: (0, 0, 0)),
            pl.BlockSpec((1, D), lambda g: (0, 0)),
            pl.BlockSpec((1, D), lambda g: (0, 0)),
            pl.BlockSpec((D, D), lambda g: (0, 0)),
            pl.BlockSpec((1, D), lambda g: (0, 0)),
            pl.BlockSpec((D, CODE_DIM), lambda g: (0, 0)),
            pl.BlockSpec((1, CODE_DIM), lambda g: (0, 0)),
            pl.BlockSpec((N_CODES, CODE_DIM), lambda g: (0, 0)),
        ],
        out_specs=pl.BlockSpec((1, B * slots), lambda g: (0, 0)),
        out_shape=jax.ShapeDtypeStruct((1, B * slots), jnp.int32),
        scratch_shapes=[pltpu.VMEM((B * slots, D), jnp.float32)],
    )(vmask, eig_vectors, xn, ln1_g.reshape(1, D), ln1_b.reshape(1, D),
      W1, b1.reshape(1, D), W2, b2.reshape(1, CODE_DIM), mu)

    idx_flat = jnp.flip(idx.reshape(B, slots), axis=1).reshape(B * slots)
    parity = (idx_flat & 1).astype(jnp.float32).reshape(B * slots, 1)

    muP, lsP = _sc_gather(mu.reshape(N_CODES // 2, 2 * CODE_DIM),
                          log_sigma.reshape(N_CODES // 2, 2 * CODE_DIM),
                          idx_flat >> 1)

    out = pl.pallas_call(
        _up_kernel,
        in_specs=[
            pl.BlockSpec((B * slots, 2 * CODE_DIM), lambda: (0, 0)),
            pl.BlockSpec((B * slots, 2 * CODE_DIM), lambda: (0, 0)),
            pl.BlockSpec((B * slots, 1), lambda: (0, 0)),
            pl.BlockSpec((B * slots, CODE_DIM), lambda: (0, 0)),
            pl.BlockSpec((CODE_DIM, D), lambda: (0, 0)),
            pl.BlockSpec((1, D), lambda: (0, 0)),
            pl.BlockSpec((D, D), lambda: (0, 0)),
            pl.BlockSpec((1, D), lambda: (0, 0)),
            pl.BlockSpec((1, D), lambda: (0, 0)),
            pl.BlockSpec((1, D), lambda: (0, 0)),
        ],
        out_specs=pl.BlockSpec((B * slots, D), lambda: (0, 0)),
        out_shape=jax.ShapeDtypeStruct((B * slots, D), jnp.float32),
    )(muP, lsP, parity, noise.reshape(B * slots, CODE_DIM), W3,
      b3.reshape(1, D), W4, b4.reshape(1, D), ln2_g.reshape(1, D),
      ln2_b.reshape(1, D))

    return out.reshape(B, slots, D)
